# Initial kernel scaffold; baseline (speedup 1.0000x reference)
import jax
import jax.numpy as jnp
from jax.experimental import pallas as pl


def kernel(query_idx, doc_idx, query_len, doc_len, emb_table, dense_w, dense_b):
    raise NotImplementedError("write your pallas kernel here")


# speedup vs baseline: 1.0927x; 1.0927x over previous
import jax
import jax.numpy as jnp
from jax.experimental import pallas as pl
from jax.experimental.pallas import tpu as pltpu

B, Q, D, E, K = 128, 32, 512, 300, 21


def _rbf_mus(n):
    mus = [1.0]
    if n == 1:
        return mus
    bin_size = 2.0 / (n - 1)
    mus.append(1 - bin_size / 2)
    for i in range(1, n - 1):
        mus.append(mus[i] - bin_size)
    return mus


def _rbf_neg_inv_two_sigma_sq(n):
    sigmas = [0.001] + [0.1] * (n - 1)
    return [-1.0 / (2.0 * s * s) for s in sigmas]


_MUS = _rbf_mus(K)
_NEG_C = _rbf_neg_inv_two_sigma_sq(K)


def _knrm_body(qe_ref, de_ref, mq_ref, md_ref, w_ref, b_ref, lps_ref, sc_ref):
    qe = qe_ref[0]
    de = de_ref[0]

    qn2 = jnp.sum(qe * qe, axis=1, keepdims=True)
    qn = qe * jax.lax.rsqrt(jnp.maximum(qn2, 1e-24))
    dn2 = jnp.sum(de * de, axis=1, keepdims=True)
    dn = de * jax.lax.rsqrt(jnp.maximum(dn2, 1e-24))

    sim = jax.lax.dot_general(
        qn, dn, (((1,), (1,)), ((), ())),
        preferred_element_type=jnp.float32)

    md = md_ref[0]
    sim = jnp.where(md > 0.0, sim, -30.0)

    sums = []
    for k in range(K):
        diff = sim - _MUS[k]
        p = jnp.exp(diff * diff * _NEG_C[k])
        sums.append(jnp.sum(p, axis=1, keepdims=True))
    ps = jnp.concatenate(sums, axis=1)

    lp = jnp.log(jnp.maximum(ps, 1e-10)) * 0.01

    mq = mq_ref[0]
    lpsum = jax.lax.dot_general(
        mq, lp, (((1,), (0,)), ((), ())),
        preferred_element_type=jnp.float32)

    lps_ref[0] = lpsum
    w = w_ref[0]
    sc_ref[0] = jnp.sum(lpsum * w, axis=1, keepdims=True) + b_ref[0]


@jax.jit
def kernel(query_idx, doc_idx, query_len, doc_len, emb_table, dense_w, dense_b):
    q_emb = emb_table[query_idx]
    d_emb = emb_table[doc_idx]
    mask_q = (jnp.arange(Q)[None, :] < query_len[:, None]).astype(jnp.float32)
    mask_d = (jnp.arange(D)[None, :] < doc_len[:, None]).astype(jnp.float32)

    lps, score = pl.pallas_call(
        _knrm_body,
        grid=(B,),
        in_specs=[
            pl.BlockSpec((1, Q, E), lambda b: (b, 0, 0)),
            pl.BlockSpec((1, D, E), lambda b: (b, 0, 0)),
            pl.BlockSpec((1, 1, Q), lambda b: (b, 0, 0)),
            pl.BlockSpec((1, 1, D), lambda b: (b, 0, 0)),
            pl.BlockSpec((1, 1, K), lambda b: (0, 0, 0)),
            pl.BlockSpec((1, 1, 1), lambda b: (0, 0, 0)),
        ],
        out_specs=(
            pl.BlockSpec((1, 1, K), lambda b: (b, 0, 0)),
            pl.BlockSpec((1, 1, 1), lambda b: (b, 0, 0)),
        ),
        out_shape=(
            jax.ShapeDtypeStruct((B, 1, K), jnp.float32),
            jax.ShapeDtypeStruct((B, 1, 1), jnp.float32),
        ),
        compiler_params=pltpu.CompilerParams(
            dimension_semantics=("parallel",),
        ),
    )(q_emb, d_emb, mask_q[:, None, :], mask_d[:, None, :],
      dense_w.reshape(1, 1, K), dense_b.reshape(1, 1, 1))

    return score[:, 0, 0], lps[:, 0, :]


# speedup vs baseline: 1.1089x; 1.0148x over previous
import jax
import jax.numpy as jnp
from jax.experimental import pallas as pl
from jax.experimental.pallas import tpu as pltpu

B, Q, D, E, K = 128, 32, 512, 300, 21


def _rbf_mus(n):
    mus = [1.0]
    if n == 1:
        return mus
    bin_size = 2.0 / (n - 1)
    mus.append(1 - bin_size / 2)
    for i in range(1, n - 1):
        mus.append(mus[i] - bin_size)
    return mus


def _rbf_neg_inv_two_sigma_sq(n):
    sigmas = [0.001] + [0.1] * (n - 1)
    return [-1.0 / (2.0 * s * s) for s in sigmas]


_MUS = _rbf_mus(K)
_NEG_C = _rbf_neg_inv_two_sigma_sq(K)


def _knrm_body(emb_ref, mq_ref, md_ref, w_ref, b_ref, lps_ref, sc_ref):
    qe = emb_ref[0, :Q, :]
    de = emb_ref[0, Q:, :]

    qn2 = jnp.sum(qe * qe, axis=1, keepdims=True)
    qn = qe * jax.lax.rsqrt(jnp.maximum(qn2, 1e-24))
    dn2 = jnp.sum(de * de, axis=1, keepdims=True)
    dn = de * jax.lax.rsqrt(jnp.maximum(dn2, 1e-24))

    sim = jax.lax.dot_general(
        qn, dn, (((1,), (1,)), ((), ())),
        preferred_element_type=jnp.float32)

    md = md_ref[0]
    sim = jnp.where(md > 0.0, sim, -30.0)

    sums = []
    for k in range(K):
        diff = sim - _MUS[k]
        p = jnp.exp(diff * diff * _NEG_C[k])
        sums.append(jnp.sum(p, axis=1, keepdims=True))
    ps = jnp.concatenate(sums, axis=1)

    lp = jnp.log(jnp.maximum(ps, 1e-10)) * 0.01

    mq = mq_ref[0]
    lpsum = jax.lax.dot_general(
        mq, lp, (((1,), (0,)), ((), ())),
        preferred_element_type=jnp.float32)

    lps_ref[0] = lpsum
    w = w_ref[0]
    sc_ref[0] = jnp.sum(lpsum * w, axis=1, keepdims=True) + b_ref[0]


@jax.jit
def kernel(query_idx, doc_idx, query_len, doc_len, emb_table, dense_w, dense_b):
    qd_emb = emb_table[jnp.concatenate([query_idx, doc_idx], axis=1)]
    mask_q = (jnp.arange(Q)[None, :] < query_len[:, None]).astype(jnp.float32)
    mask_d = (jnp.arange(D)[None, :] < doc_len[:, None]).astype(jnp.float32)

    lps, score = pl.pallas_call(
        _knrm_body,
        grid=(B,),
        in_specs=[
            pl.BlockSpec((1, Q + D, E), lambda b: (b, 0, 0)),
            pl.BlockSpec((1, 1, Q), lambda b: (b, 0, 0)),
            pl.BlockSpec((1, 1, D), lambda b: (b, 0, 0)),
            pl.BlockSpec((1, 1, K), lambda b: (0, 0, 0)),
            pl.BlockSpec((1, 1, 1), lambda b: (0, 0, 0)),
        ],
        out_specs=(
            pl.BlockSpec((1, 1, K), lambda b: (b, 0, 0)),
            pl.BlockSpec((1, 1, 1), lambda b: (b, 0, 0)),
        ),
        out_shape=(
            jax.ShapeDtypeStruct((B, 1, K), jnp.float32),
            jax.ShapeDtypeStruct((B, 1, 1), jnp.float32),
        ),
        compiler_params=pltpu.CompilerParams(
            dimension_semantics=("parallel",),
        ),
    )(qd_emb, mask_q[:, None, :], mask_d[:, None, :],
      dense_w.reshape(1, 1, K), dense_b.reshape(1, 1, 1))

    return score[:, 0, 0], lps[:, 0, :]


# speedup vs baseline: 1.1222x; 1.0120x over previous
import jax
import jax.numpy as jnp
from jax.experimental import pallas as pl
from jax.experimental.pallas import tpu as pltpu

B, Q, D, E, K = 128, 32, 512, 300, 21


def _rbf_mus(n):
    mus = [1.0]
    if n == 1:
        return mus
    bin_size = 2.0 / (n - 1)
    mus.append(1 - bin_size / 2)
    for i in range(1, n - 1):
        mus.append(mus[i] - bin_size)
    return mus


def _rbf_neg_inv_two_sigma_sq(n):
    sigmas = [0.001] + [0.1] * (n - 1)
    return [-1.0 / (2.0 * s * s) for s in sigmas]


_MUS = _rbf_mus(K)
_NEG_C = _rbf_neg_inv_two_sigma_sq(K)


def _knrm_body(emb_ref, ql_ref, dl_ref, w_ref, b_ref, lps_ref, sc_ref):
    qe = emb_ref[0, :Q, :]
    de = emb_ref[0, Q:, :]

    qn2 = jnp.sum(qe * qe, axis=1, keepdims=True)
    qn = qe * jax.lax.rsqrt(jnp.maximum(qn2, 1e-24))
    dn2 = jnp.sum(de * de, axis=1, keepdims=True)
    dn = de * jax.lax.rsqrt(jnp.maximum(dn2, 1e-24))

    sim = jax.lax.dot_general(
        qn.astype(jnp.bfloat16), dn.astype(jnp.bfloat16),
        (((1,), (1,)), ((), ())),
        preferred_element_type=jnp.float32)

    dlen = dl_ref[0, 0, 0]
    dmask = jax.lax.broadcasted_iota(jnp.int32, (Q, D), 1) < dlen
    sim = jnp.where(dmask, sim, -30.0)

    sums = []
    for k in range(K):
        diff = sim - _MUS[k]
        p = jnp.exp(diff * diff * _NEG_C[k])
        sums.append(jnp.sum(p, axis=1, keepdims=True))
    ps = jnp.concatenate(sums, axis=1)

    lp = jnp.log(jnp.maximum(ps, 1e-10)) * 0.01

    qlen = ql_ref[0, 0, 0]
    qmask = jax.lax.broadcasted_iota(jnp.int32, (Q, K), 0) < qlen
    lpsum = jnp.sum(jnp.where(qmask, lp, 0.0), axis=0, keepdims=True)

    lps_ref[0] = lpsum

    wb = w_ref[0].astype(jnp.bfloat16).astype(jnp.float32)
    lb = lpsum.astype(jnp.bfloat16).astype(jnp.float32)
    sc_ref[0] = jnp.sum(lb * wb, axis=1, keepdims=True) + b_ref[0]


@jax.jit
def kernel(query_idx, doc_idx, query_len, doc_len, emb_table, dense_w, dense_b):
    qd_emb = emb_table[jnp.concatenate([query_idx, doc_idx], axis=1)]

    lps, score = pl.pallas_call(
        _knrm_body,
        grid=(B,),
        in_specs=[
            pl.BlockSpec((1, Q + D, E), lambda b: (b, 0, 0)),
            pl.BlockSpec((1, 1, 1), lambda b: (b, 0, 0), memory_space=pltpu.SMEM),
            pl.BlockSpec((1, 1, 1), lambda b: (b, 0, 0), memory_space=pltpu.SMEM),
            pl.BlockSpec((1, 1, K), lambda b: (0, 0, 0)),
            pl.BlockSpec((1, 1, 1), lambda b: (0, 0, 0)),
        ],
        out_specs=(
            pl.BlockSpec((1, 1, K), lambda b: (b, 0, 0)),
            pl.BlockSpec((1, 1, 1), lambda b: (b, 0, 0)),
        ),
        out_shape=(
            jax.ShapeDtypeStruct((B, 1, K), jnp.float32),
            jax.ShapeDtypeStruct((B, 1, 1), jnp.float32),
        ),
        compiler_params=pltpu.CompilerParams(
            dimension_semantics=("parallel",),
        ),
    )(qd_emb, query_len.reshape(B, 1, 1), doc_len.reshape(B, 1, 1),
      dense_w.reshape(1, 1, K), dense_b.reshape(1, 1, 1))

    return score[:, 0, 0], lps[:, 0, :]
